# baseline (speedup 1.0000x reference)
import functools

import jax
import jax.numpy as jnp
from jax import lax
from jax.experimental import pallas as pl
from jax.experimental.pallas import tpu as pltpu
from jax.experimental.pallas import tpu_sc as plsc

_TILES = 16
_CORES = 2
_C = 80


def _deg_body(K, S, N_pad,
              cidx4, ewf4, zerosN, deg_out, cv, wv, deg_sh0, deg_sh1, dsem):
    cid = lax.axis_index("c")
    sid = lax.axis_index("s")

    @pl.when(sid == 0)
    def _zero():
        pltpu.sync_copy(zerosN, deg_sh0)
        pltpu.sync_copy(zerosN, deg_sh1)

    plsc.subcore_barrier()

    for si in range(S // _CORES):
        deg_sh = (deg_sh0, deg_sh1)[si]
        sc = 2 * cid + si
        pltpu.sync_copy(cidx4.at[sc, sid], cv)
        pltpu.sync_copy(ewf4.at[sc, sid], wv)

        def fire(k, carry):
            pltpu.async_copy(wv.at[k], deg_sh.at[cv.at[k]], dsem, add=True)
            return carry

        lax.fori_loop(0, K, fire, 0)

        def drain(k, carry):
            pltpu.make_async_copy(wv.at[0], deg_sh.at[cv.at[0]], dsem).wait()
            return carry

        lax.fori_loop(0, K, drain, 0)

    plsc.subcore_barrier()

    @pl.when(sid == 0)
    def _writeout():
        for si in range(S // _CORES):
            deg_sh = (deg_sh0, deg_sh1)[si]
            sc = 2 * cid + si
            pltpu.sync_copy(deg_sh, deg_out.at[pl.ds(sc * N_pad, N_pad)])


def _agg_body(K, S, N, H,
              edi, ewf4, hs, zerosNH, acc_out,
              edv0, edv1, ewv0, ewv1, rows0, rows1, rows2, rows3, acc_sh,
              gs0, gs1, gs2, gs3, ss0, ss1, ss2, ss3):
    cid = lax.axis_index("c")
    sid = lax.axis_index("s")
    N_pad = zerosNH.shape[0]
    rpt = N_pad // _TILES
    n_full = N // rpt
    rem = N - n_full * rpt
    NBP = K // 16
    rows = (rows0, rows1, rows2, rows3)
    gs = (gs0, gs1, gs2, gs3)
    ss = (ss0, ss1, ss2, ss3)

    for si in range(S // _CORES):
        sc = 2 * cid + si
        zsl = pl.ds(sid * rpt, rpt)
        pltpu.sync_copy(zerosNH.at[zsl], acc_sh.at[zsl])
        plsc.subcore_barrier()

        pltpu.sync_copy(edi.at[sc, sid, pl.ds(0, 16)], edv0)
        pltpu.sync_copy(ewf4.at[sc, sid, pl.ds(0, 8)], ewv0.at[pl.ds(0, 8)])
        pltpu.async_copy(hs.at[edv0.at[0]], rows0, gs0)
        pltpu.async_copy(hs.at[edv0.at[2]], rows1, gs1)

        def _mul(EW, j, rows_p):
            def rowmul(r, carry):
                wv = EW[j, pl.ds(r, 16)]
                wf = jnp.full((16,), wv[0], jnp.float32)
                for jj in range(H // 16):
                    sl = pl.ds(jj * 16, 16)
                    rows_p[r, sl] = rows_p[r, sl] * wf
                return carry

            lax.fori_loop(0, _C, rowmul, 0, unroll=8)

        def _chunk(E, Enext, EW, j, wait_guard, fire_guard):
            p = j % 4
            pn = (j + 2) % 4
            j2 = 2 * j

            def _wait_old():
                pltpu.make_async_copy(
                    rows[pn], acc_sh.at[E.at[j2 + 1]], ss[pn]).wait()

            if wait_guard is None:
                _wait_old()
            else:
                pl.when(wait_guard)(_wait_old)

            if j < 6:
                nidx = E.at[j2 + 4]
            else:
                nidx = Enext.at[2 * (j - 6)]

            def _fire_next():
                pltpu.async_copy(hs.at[nidx], rows[pn], gs[pn])

            if fire_guard is None:
                _fire_next()
            else:
                pl.when(fire_guard)(_fire_next)

            pltpu.make_async_copy(hs.at[E.at[j2]], rows[p], gs[p]).wait()
            _mul(EW, j, rows[p])
            pltpu.async_copy(
                rows[p], acc_sh.at[E.at[j2 + 1]], ss[p], add=True)

        def pair(i, carry):
            pltpu.sync_copy(edi.at[sc, sid, pl.ds((2 * i + 1) * 16, 16)],
                            edv1)
            pltpu.sync_copy(ewf4.at[sc, sid, pl.ds((2 * i + 1) * 8, 8)],
                            ewv1.at[pl.ds(0, 8)])
            for j in range(8):
                _chunk(edv0, edv1, ewv0, j,
                       wait_guard=(i > 0) if j < 2 else None,
                       fire_guard=None)

            @pl.when(i < NBP - 1)
            def _prefetch_a2():
                pltpu.sync_copy(
                    edi.at[sc, sid, pl.ds((2 * i + 2) * 16, 16)], edv0)
                pltpu.sync_copy(
                    ewf4.at[sc, sid, pl.ds((2 * i + 2) * 8, 8)],
                    ewv0.at[pl.ds(0, 8)])

            for j in range(8):
                _chunk(edv1, edv0, ewv1, j,
                       wait_guard=None,
                       fire_guard=(i < NBP - 1) if j >= 6 else None)
            return carry

        lax.fori_loop(0, NBP, pair, 0)
        for d in (2, 1):
            sl = (K - d) % 4
            pltpu.make_async_copy(rows[sl], acc_sh.at[edv1.at[1]],
                                  ss[sl]).wait()
        plsc.subcore_barrier()

        @pl.when(sid < n_full)
        def _wfull():
            sl = pl.ds(sid * rpt, rpt)
            pltpu.sync_copy(acc_sh.at[sl], acc_out.at[sc, sl])

        if rem:
            @pl.when(sid == n_full)
            def _wrem():
                sl = pl.ds(n_full * rpt, rem)
                pltpu.sync_copy(acc_sh.at[sl], acc_out.at[sc, sl])

        plsc.subcore_barrier()


def _mm1_body(x_ref, w_ref, deg_ref, hs_ref, dis_ref):
    S = w_ref.shape[0]
    dis = lax.rsqrt(deg_ref[...] + 1.0)
    dis_ref[...] = dis
    for s in range(S):
        h = jnp.dot(x_ref[...], w_ref[s], preferred_element_type=jnp.float32)
        hs_ref[s] = h * dis[s][:, None]


def _mm2_body(acc_ref, hs_ref, dis_ref, b_ref, a_ref, w_ref, hs2_ref):
    S = w_ref.shape[0]
    for s in range(S):
        dis = dis_ref[s][:, None]
        z = (acc_ref[s] + hs_ref[s]) * dis + b_ref[s][None, :]
        z = jnp.where(z > 0, z, a_ref[s][None, :] * z)
        h2 = jnp.dot(z, w_ref[s], preferred_element_type=jnp.float32)
        hs2_ref[s] = h2 * dis


def _fin_body(acc_ref, hs_ref, dis_ref, b_ref, a_ref, out_ref):
    S = acc_ref.shape[0]
    for s in range(S):
        z = (acc_ref[s] + hs_ref[s]) * dis_ref[s][:, None] + b_ref[s][None, :]
        out_ref[:, s, :] = jnp.where(z > 0, z, a_ref[s][None, :] * z)


def kernel(x, edge_index, edge_weight, W1, b1, W2, b2, prelu_a):
    N, D = x.shape
    S, _, E = edge_index.shape
    H = W1.shape[2]

    K0 = -(-E // (_TILES * _C))
    K = -(-K0 // 16) * 16
    E_pad = _TILES * K * _C
    pad = E_pad - E

    row = edge_index[:, 0, :]
    col = edge_index[:, 1, :]
    pidx = (jnp.arange(pad, dtype=jnp.int32) * 17) % N
    row_p = jnp.concatenate([row, jnp.broadcast_to(pidx, (S, pad))], axis=1)
    col_p = jnp.concatenate([col, jnp.broadcast_to(pidx, (S, pad))], axis=1)
    ew_p = jnp.concatenate(
        [edge_weight, jnp.zeros((S, pad), jnp.float32)], axis=1)
    row4 = (row_p + (jnp.arange(S, dtype=jnp.int32) * N)[:, None]).reshape(
        S, _TILES, K, _C)
    col4 = col_p.reshape(S, _TILES, K, _C)
    edi = jnp.stack([row4, col4], axis=3).reshape(S, _TILES, 2 * K, _C)
    cidx4 = col4
    ewf4 = ew_p.reshape(S, _TILES, K, _C)
    rpt = -(-N // (_TILES * 8)) * 8
    N_pad = _TILES * rpt
    assert N_pad % 128 == 0
    zerosN = jnp.zeros((N_pad,), jnp.float32)
    zerosNH = jnp.zeros((N_pad, H), jnp.float32)

    mesh = plsc.VectorSubcoreMesh(core_axis_name="c", subcore_axis_name="s")

    deg = pl.kernel(
        functools.partial(_deg_body, K, S, N_pad),
        out_type=jax.ShapeDtypeStruct((S * N_pad,), jnp.float32),
        mesh=mesh,
        scratch_types=[
            pltpu.VMEM((K, _C), jnp.int32),
            pltpu.VMEM((K, _C), jnp.float32),
            pltpu.VMEM_SHARED((N_pad,), jnp.float32),
            pltpu.VMEM_SHARED((N_pad,), jnp.float32),
            pltpu.SemaphoreType.DMA,
        ],
    )(cidx4, ewf4, zerosN).reshape(S, N_pad)[:, :N]

    agg = pl.kernel(
        functools.partial(_agg_body, K, S, N, H),
        out_type=jax.ShapeDtypeStruct((S, N, H), jnp.float32),
        mesh=mesh,
        scratch_types=[
            pltpu.VMEM((16, _C), jnp.int32),
            pltpu.VMEM((16, _C), jnp.int32),
            pltpu.VMEM((9, _C), jnp.float32),
            pltpu.VMEM((9, _C), jnp.float32),
            pltpu.VMEM((_C, H), jnp.float32),
            pltpu.VMEM((_C, H), jnp.float32),
            pltpu.VMEM((_C, H), jnp.float32),
            pltpu.VMEM((_C, H), jnp.float32),
            pltpu.VMEM_SHARED((N_pad, H), jnp.float32),
        ] + [pltpu.SemaphoreType.DMA] * 8,
    )

    TN = 1024
    NT = -(-N // TN)
    grid = (NT,)

    hs1, dis = pl.pallas_call(
        _mm1_body,
        grid=grid,
        in_specs=[
            pl.BlockSpec((TN, D), lambda n: (n, 0)),
            pl.BlockSpec((S, D, H), lambda n: (0, 0, 0)),
            pl.BlockSpec((S, TN), lambda n: (0, n)),
        ],
        out_specs=[
            pl.BlockSpec((S, TN, H), lambda n: (0, n, 0)),
            pl.BlockSpec((S, TN), lambda n: (0, n)),
        ],
        out_shape=[
            jax.ShapeDtypeStruct((S, N, H), jnp.float32),
            jax.ShapeDtypeStruct((S, N), jnp.float32),
        ],
    )(x, W1, deg)

    acc1 = agg(edi, ewf4, hs1.reshape(S * N, H), zerosNH)

    hs2 = pl.pallas_call(
        _mm2_body,
        grid=grid,
        in_specs=[
            pl.BlockSpec((S, TN, H), lambda n: (0, n, 0)),
            pl.BlockSpec((S, TN, H), lambda n: (0, n, 0)),
            pl.BlockSpec((S, TN), lambda n: (0, n)),
            pl.BlockSpec((S, H), lambda n: (0, 0)),
            pl.BlockSpec((S, H), lambda n: (0, 0)),
            pl.BlockSpec((S, H, H), lambda n: (0, 0, 0)),
        ],
        out_specs=pl.BlockSpec((S, TN, H), lambda n: (0, n, 0)),
        out_shape=jax.ShapeDtypeStruct((S, N, H), jnp.float32),
    )(acc1, hs1, dis, b1, prelu_a, W2)

    acc2 = agg(edi, ewf4, hs2.reshape(S * N, H), zerosNH)

    out = pl.pallas_call(
        _fin_body,
        grid=grid,
        in_specs=[
            pl.BlockSpec((S, TN, H), lambda n: (0, n, 0)),
            pl.BlockSpec((S, TN, H), lambda n: (0, n, 0)),
            pl.BlockSpec((S, TN), lambda n: (0, n)),
            pl.BlockSpec((S, H), lambda n: (0, 0)),
            pl.BlockSpec((S, H), lambda n: (0, 0)),
        ],
        out_specs=pl.BlockSpec((TN, S, H), lambda n: (n, 0, 0)),
        out_shape=jax.ShapeDtypeStruct((N, S, H), jnp.float32),
    )(acc2, hs2, dis, b2, prelu_a)

    return out


# speedup vs baseline: 1.0623x; 1.0623x over previous
import functools

import jax
import jax.numpy as jnp
from jax import lax
from jax.experimental import pallas as pl
from jax.experimental.pallas import tpu as pltpu
from jax.experimental.pallas import tpu_sc as plsc

_TILES = 16
_CORES = 2
_C = 80


def _deg_body(K, S, N_pad,
              cidx4, ewf4, zerosN, deg_out, cv, wv, deg_sh0, deg_sh1, dsem):
    cid = lax.axis_index("c")
    sid = lax.axis_index("s")

    @pl.when(sid == 0)
    def _zero():
        pltpu.sync_copy(zerosN, deg_sh0)
        pltpu.sync_copy(zerosN, deg_sh1)

    plsc.subcore_barrier()

    for si in range(S // _CORES):
        deg_sh = (deg_sh0, deg_sh1)[si]
        sc = 2 * cid + si
        pltpu.sync_copy(cidx4.at[sc, sid], cv)
        pltpu.sync_copy(ewf4.at[sc, sid], wv)

        def fire(k, carry):
            pltpu.async_copy(wv.at[k], deg_sh.at[cv.at[k]], dsem, add=True)
            return carry

        lax.fori_loop(0, K, fire, 0)

        def drain(k, carry):
            pltpu.make_async_copy(wv.at[0], deg_sh.at[cv.at[0]], dsem).wait()
            return carry

        lax.fori_loop(0, K, drain, 0)

    plsc.subcore_barrier()

    @pl.when(sid == 0)
    def _writeout():
        for si in range(S // _CORES):
            deg_sh = (deg_sh0, deg_sh1)[si]
            sc = 2 * cid + si
            pltpu.sync_copy(deg_sh, deg_out.at[pl.ds(sc * N_pad, N_pad)])


def _agg_body(K, S, N, H,
              edi, ewf4, hs, zerosNH, acc_out,
              edv0, edv1, ewv0, ewv1, rows0, rows1, rows2, rows3, acc_sh,
              gs0, gs1, gs2, gs3, ss0, ss1, ss2, ss3):
    cid = lax.axis_index("c")
    sid = lax.axis_index("s")
    N_pad = zerosNH.shape[0]
    rpt = N_pad // _TILES
    n_full = N // rpt
    rem = N - n_full * rpt
    NBP = K // 16
    rows = (rows0, rows1, rows2, rows3)
    gs = (gs0, gs1, gs2, gs3)
    ss = (ss0, ss1, ss2, ss3)

    for si in range(S // _CORES):
        sc = 2 * cid + si
        zsl = pl.ds(sid * rpt, rpt)
        pltpu.sync_copy(zerosNH.at[zsl], acc_sh.at[zsl])
        plsc.subcore_barrier()

        pltpu.sync_copy(edi.at[sc, sid, pl.ds(0, 16)], edv0)
        pltpu.sync_copy(ewf4.at[sc, sid, pl.ds(0, 8)], ewv0.at[pl.ds(0, 8)])
        pltpu.async_copy(hs.at[edv0.at[0]], rows0, gs0)
        pltpu.async_copy(hs.at[edv0.at[2]], rows1, gs1)

        def _mul(EW, j, rows_p):
            def rowmul(r, carry):
                wv = EW[j, pl.ds(r, 16)]
                wf = jnp.full((16,), wv[0], jnp.float32)
                for jj in range(H // 16):
                    sl = pl.ds(jj * 16, 16)
                    rows_p[r, sl] = rows_p[r, sl] * wf
                return carry

            lax.fori_loop(0, _C, rowmul, 0, unroll=4)

        def _chunk(E, Enext, EW, j, wait_guard, fire_guard):
            p = j % 4
            pn = (j + 2) % 4
            j2 = 2 * j

            def _wait_old():
                pltpu.make_async_copy(
                    rows[pn], acc_sh.at[E.at[j2 + 1]], ss[pn]).wait()

            if wait_guard is None:
                _wait_old()
            else:
                pl.when(wait_guard)(_wait_old)

            if j < 6:
                nidx = E.at[j2 + 4]
            else:
                nidx = Enext.at[2 * (j - 6)]

            def _fire_next():
                pltpu.async_copy(hs.at[nidx], rows[pn], gs[pn])

            if fire_guard is None:
                _fire_next()
            else:
                pl.when(fire_guard)(_fire_next)

            pltpu.make_async_copy(hs.at[E.at[j2]], rows[p], gs[p]).wait()
            _mul(EW, j, rows[p])
            pltpu.async_copy(
                rows[p], acc_sh.at[E.at[j2 + 1]], ss[p], add=True)

        def pair(i, carry):
            pltpu.sync_copy(edi.at[sc, sid, pl.ds((2 * i + 1) * 16, 16)],
                            edv1)
            pltpu.sync_copy(ewf4.at[sc, sid, pl.ds((2 * i + 1) * 8, 8)],
                            ewv1.at[pl.ds(0, 8)])
            for j in range(8):
                _chunk(edv0, edv1, ewv0, j,
                       wait_guard=(i > 0) if j < 2 else None,
                       fire_guard=None)

            @pl.when(i < NBP - 1)
            def _prefetch_a2():
                pltpu.sync_copy(
                    edi.at[sc, sid, pl.ds((2 * i + 2) * 16, 16)], edv0)
                pltpu.sync_copy(
                    ewf4.at[sc, sid, pl.ds((2 * i + 2) * 8, 8)],
                    ewv0.at[pl.ds(0, 8)])

            for j in range(8):
                _chunk(edv1, edv0, ewv1, j,
                       wait_guard=None,
                       fire_guard=(i < NBP - 1) if j >= 6 else None)
            return carry

        lax.fori_loop(0, NBP, pair, 0)
        for d in (2, 1):
            sl = (K - d) % 4
            pltpu.make_async_copy(rows[sl], acc_sh.at[edv1.at[1]],
                                  ss[sl]).wait()
        plsc.subcore_barrier()

        @pl.when(sid < n_full)
        def _wfull():
            sl = pl.ds(sid * rpt, rpt)
            pltpu.sync_copy(acc_sh.at[sl], acc_out.at[sc, sl])

        if rem:
            @pl.when(sid == n_full)
            def _wrem():
                sl = pl.ds(n_full * rpt, rem)
                pltpu.sync_copy(acc_sh.at[sl], acc_out.at[sc, sl])

        plsc.subcore_barrier()


def _mm1_body(x_ref, w_ref, deg_ref, hs_ref, dis_ref):
    S = w_ref.shape[0]
    dis = lax.rsqrt(deg_ref[...] + 1.0)
    dis_ref[...] = dis
    for s in range(S):
        h = jnp.dot(x_ref[...], w_ref[s], preferred_element_type=jnp.float32)
        hs_ref[s] = h * dis[s][:, None]


def _mm2_body(acc_ref, hs_ref, dis_ref, b_ref, a_ref, w_ref, hs2_ref):
    S = w_ref.shape[0]
    for s in range(S):
        dis = dis_ref[s][:, None]
        z = (acc_ref[s] + hs_ref[s]) * dis + b_ref[s][None, :]
        z = jnp.where(z > 0, z, a_ref[s][None, :] * z)
        h2 = jnp.dot(z, w_ref[s], preferred_element_type=jnp.float32)
        hs2_ref[s] = h2 * dis


def _fin_body(acc_ref, hs_ref, dis_ref, b_ref, a_ref, out_ref):
    S = acc_ref.shape[0]
    for s in range(S):
        z = (acc_ref[s] + hs_ref[s]) * dis_ref[s][:, None] + b_ref[s][None, :]
        out_ref[:, s, :] = jnp.where(z > 0, z, a_ref[s][None, :] * z)


def kernel(x, edge_index, edge_weight, W1, b1, W2, b2, prelu_a):
    N, D = x.shape
    S, _, E = edge_index.shape
    H = W1.shape[2]

    K0 = -(-E // (_TILES * _C))
    K = -(-K0 // 16) * 16
    E_pad = _TILES * K * _C
    pad = E_pad - E

    row = edge_index[:, 0, :]
    col = edge_index[:, 1, :]
    pidx = (jnp.arange(pad, dtype=jnp.int32) * 17) % N
    row_p = jnp.concatenate([row, jnp.broadcast_to(pidx, (S, pad))], axis=1)
    col_p = jnp.concatenate([col, jnp.broadcast_to(pidx, (S, pad))], axis=1)
    ew_p = jnp.concatenate(
        [edge_weight, jnp.zeros((S, pad), jnp.float32)], axis=1)
    row4 = (row_p + (jnp.arange(S, dtype=jnp.int32) * N)[:, None]).reshape(
        S, _TILES, K, _C)
    col4 = col_p.reshape(S, _TILES, K, _C)
    edi = jnp.stack([row4, col4], axis=3).reshape(S, _TILES, 2 * K, _C)
    cidx4 = col4
    ewf4 = ew_p.reshape(S, _TILES, K, _C)
    rpt = -(-N // (_TILES * 8)) * 8
    N_pad = _TILES * rpt
    assert N_pad % 128 == 0
    zerosN = jnp.zeros((N_pad,), jnp.float32)
    zerosNH = jnp.zeros((N_pad, H), jnp.float32)

    mesh = plsc.VectorSubcoreMesh(core_axis_name="c", subcore_axis_name="s")

    deg = pl.kernel(
        functools.partial(_deg_body, K, S, N_pad),
        out_type=jax.ShapeDtypeStruct((S * N_pad,), jnp.float32),
        mesh=mesh,
        scratch_types=[
            pltpu.VMEM((K, _C), jnp.int32),
            pltpu.VMEM((K, _C), jnp.float32),
            pltpu.VMEM_SHARED((N_pad,), jnp.float32),
            pltpu.VMEM_SHARED((N_pad,), jnp.float32),
            pltpu.SemaphoreType.DMA,
        ],
    )(cidx4, ewf4, zerosN).reshape(S, N_pad)[:, :N]

    agg = pl.kernel(
        functools.partial(_agg_body, K, S, N, H),
        out_type=jax.ShapeDtypeStruct((S, N, H), jnp.float32),
        mesh=mesh,
        scratch_types=[
            pltpu.VMEM((16, _C), jnp.int32),
            pltpu.VMEM((16, _C), jnp.int32),
            pltpu.VMEM((9, _C), jnp.float32),
            pltpu.VMEM((9, _C), jnp.float32),
            pltpu.VMEM((_C, H), jnp.float32),
            pltpu.VMEM((_C, H), jnp.float32),
            pltpu.VMEM((_C, H), jnp.float32),
            pltpu.VMEM((_C, H), jnp.float32),
            pltpu.VMEM_SHARED((N_pad, H), jnp.float32),
        ] + [pltpu.SemaphoreType.DMA] * 8,
    )

    TN = 1024
    NT = -(-N // TN)
    grid = (NT,)

    hs1, dis = pl.pallas_call(
        _mm1_body,
        grid=grid,
        in_specs=[
            pl.BlockSpec((TN, D), lambda n: (n, 0)),
            pl.BlockSpec((S, D, H), lambda n: (0, 0, 0)),
            pl.BlockSpec((S, TN), lambda n: (0, n)),
        ],
        out_specs=[
            pl.BlockSpec((S, TN, H), lambda n: (0, n, 0)),
            pl.BlockSpec((S, TN), lambda n: (0, n)),
        ],
        out_shape=[
            jax.ShapeDtypeStruct((S, N, H), jnp.float32),
            jax.ShapeDtypeStruct((S, N), jnp.float32),
        ],
    )(x, W1, deg)

    acc1 = agg(edi, ewf4, hs1.reshape(S * N, H), zerosNH)

    hs2 = pl.pallas_call(
        _mm2_body,
        grid=grid,
        in_specs=[
            pl.BlockSpec((S, TN, H), lambda n: (0, n, 0)),
            pl.BlockSpec((S, TN, H), lambda n: (0, n, 0)),
            pl.BlockSpec((S, TN), lambda n: (0, n)),
            pl.BlockSpec((S, H), lambda n: (0, 0)),
            pl.BlockSpec((S, H), lambda n: (0, 0)),
            pl.BlockSpec((S, H, H), lambda n: (0, 0, 0)),
        ],
        out_specs=pl.BlockSpec((S, TN, H), lambda n: (0, n, 0)),
        out_shape=jax.ShapeDtypeStruct((S, N, H), jnp.float32),
    )(acc1, hs1, dis, b1, prelu_a, W2)

    acc2 = agg(edi, ewf4, hs2.reshape(S * N, H), zerosNH)

    out = pl.pallas_call(
        _fin_body,
        grid=grid,
        in_specs=[
            pl.BlockSpec((S, TN, H), lambda n: (0, n, 0)),
            pl.BlockSpec((S, TN, H), lambda n: (0, n, 0)),
            pl.BlockSpec((S, TN), lambda n: (0, n)),
            pl.BlockSpec((S, H), lambda n: (0, 0)),
            pl.BlockSpec((S, H), lambda n: (0, 0)),
        ],
        out_specs=pl.BlockSpec((TN, S, H), lambda n: (n, 0, 0)),
        out_shape=jax.ShapeDtypeStruct((N, S, H), jnp.float32),
    )(acc2, hs2, dis, b2, prelu_a)

    return out
